# Initial kernel scaffold; baseline (speedup 1.0000x reference)
import jax
import jax.numpy as jnp
from jax.experimental import pallas as pl


def kernel(x_obs, pos_obs, pos_query, W_enc1, b_enc1, W_enc2, b_enc2, W_qpos, b_qpos, W_k, W_v, w_dist, W_dec1, b_dec1, W_dec2, b_dec2):
    raise NotImplementedError("write your pallas kernel here")


# speedup vs baseline: 1.7028x; 1.7028x over previous
import functools

import jax
import jax.numpy as jnp
from jax.experimental import pallas as pl
from jax.experimental.pallas import tpu as pltpu

B, NO, NQ = 4, 1024, 1024
DIN, P, D, DOUT = 128, 3, 256, 128


def _dot(a, b):
    return jax.lax.dot_general(
        a, b, (((1,), (0,)), ((), ())), preferred_element_type=jnp.float32)


def _dot_nt(a, b):
    return jax.lax.dot_general(
        a, b, (((1,), (1,)), ((), ())), preferred_element_type=jnp.float32)


def _gano_body(x_ref, pot_ref, pq_ref, We1_ref, be1_ref, We2_ref, be2_ref,
               Wq_ref, bq_ref, Wk_ref, Wv_ref, wd_ref, Wd1_ref, bd1_ref,
               Wd2_ref, bd2_ref, out_ref):
    x = x_ref[0]
    h1 = jnp.maximum(_dot(x, We1_ref[...]) + be1_ref[...][None, :], 0.0)
    h = _dot(h1, We2_ref[...]) + be2_ref[...][None, :]
    k = _dot(h, Wk_ref[...])
    v = _dot(h, Wv_ref[...])
    pq = pq_ref[0]
    q = jnp.broadcast_to(bq_ref[...][None, :], (NQ, D))
    for p in range(P):
        q = q + pq[:, p:p + 1] * Wq_ref[p, :][None, :]
    pot = pot_ref[0]
    bias = jnp.zeros((1, NO), jnp.float32)
    for p in range(P):
        w_op = wd_ref[P + p] - wd_ref[2 * P + p]
        bias = bias + w_op * pot[p:p + 1, :]
    logits = _dot_nt(q, k) * (1.0 / 16.0) + bias
    m = jnp.max(logits, axis=-1, keepdims=True)
    e = jnp.exp(logits - m)
    s = jnp.sum(e, axis=-1, keepdims=True)
    attn = e / s
    hq = _dot(attn, v)
    d1 = jnp.maximum(_dot(hq, Wd1_ref[...]) + bd1_ref[...][None, :], 0.0)
    out_ref[0] = _dot(d1, Wd2_ref[...]) + bd2_ref[...][None, :]


@functools.partial(jax.jit, static_argnames=("interpret",))
def kernel(x_obs, pos_obs, pos_query, W_enc1, b_enc1, W_enc2, b_enc2,
           W_qpos, b_qpos, W_k, W_v, w_dist, W_dec1, b_dec1, W_dec2, b_dec2,
           interpret=False):
    pos_obs_t = jnp.swapaxes(pos_obs, 1, 2)
    full = lambda shape: pl.BlockSpec(shape, lambda b: (0,) * len(shape))
    grid_spec = pltpu.PrefetchScalarGridSpec(
        num_scalar_prefetch=0,
        grid=(B,),
        in_specs=[
            pl.BlockSpec((1, NO, DIN), lambda b: (b, 0, 0)),
            pl.BlockSpec((1, P, NO), lambda b: (b, 0, 0)),
            pl.BlockSpec((1, NQ, P), lambda b: (b, 0, 0)),
            full((DIN, D)), full((D,)), full((D, D)), full((D,)),
            full((P, D)), full((D,)), full((D, D)), full((D, D)),
            pl.BlockSpec(memory_space=pltpu.SMEM),
            full((D, D)), full((D,)), full((D, DOUT)), full((DOUT,)),
        ],
        out_specs=pl.BlockSpec((1, NQ, DOUT), lambda b: (b, 0, 0)),
    )
    return pl.pallas_call(
        _gano_body,
        grid_spec=grid_spec,
        out_shape=jax.ShapeDtypeStruct((B, NQ, DOUT), jnp.float32),
        interpret=interpret,
    )(x_obs, pos_obs_t, pos_query, W_enc1, b_enc1, W_enc2, b_enc2,
      W_qpos, b_qpos, W_k, W_v, w_dist, W_dec1, b_dec1, W_dec2, b_dec2)
